# baseline (speedup 1.0000x reference)
import functools

import jax
import jax.numpy as jnp
from jax import lax
from jax.experimental import pallas as pl
from jax.experimental.pallas import tpu as pltpu
from jax.experimental.pallas import tpu_sc as plsc

G = 2
NV = 320
D = 1024
VD = 512
VDH = VD // 2

BLK = 512


def _mlp_argmax_body(x_ref, w1_ref, b1_ref, w2_ref, b2_ref, idx_ref):
    x = x_ref[...]
    h = lax.dot_general(x, w1_ref[...], (((1,), (1,)), ((), ())),
                        preferred_element_type=jnp.float32)
    h = h + b1_ref[...]
    h = h * 0.5 * (1.0 + lax.erf(h * jnp.float32(0.7071067811865476)))
    logits = lax.dot_general(h, w2_ref[...], (((1,), (1,)), ((), ())),
                             preferred_element_type=jnp.float32)
    logits = logits + b2_ref[...]
    col = lax.broadcasted_iota(jnp.int32, (BLK, G * NV), 1)
    neg = jnp.float32(-jnp.inf)
    big = jnp.int32(2**30)
    outs = []
    for g in range(G):
        mask = (col >= g * NV) & (col < (g + 1) * NV)
        m = jnp.max(jnp.where(mask, logits, neg), axis=1, keepdims=True)
        hit = (logits == m) & mask
        outs.append(jnp.min(jnp.where(hit, col, big), axis=1))
    idx_ref[0] = jnp.stack(outs)


def _tc_mlp_argmax(xf, w1, b1, w2, b2):
    n = xf.shape[0]
    nblk = n // BLK
    return pl.pallas_call(
        _mlp_argmax_body,
        grid=(nblk,),
        in_specs=[
            pl.BlockSpec((BLK, D), lambda i: (i, 0)),
            pl.BlockSpec((D, D), lambda i: (0, 0)),
            pl.BlockSpec((1, D), lambda i: (0, 0)),
            pl.BlockSpec((G * NV, D), lambda i: (0, 0)),
            pl.BlockSpec((1, G * NV), lambda i: (0, 0)),
        ],
        out_specs=pl.BlockSpec((1, G, BLK), lambda i: (i, 0, 0)),
        out_shape=jax.ShapeDtypeStruct((nblk, G, BLK), jnp.int32),
    )(xf, w1, b1, w2, b2)


def _make_sc_gather(n_tok):
    info = plsc.get_sparse_core_info()
    nw = info.num_cores * info.num_subcores
    tok_per_w = n_tok // nw
    chunk = 64
    ntch = tok_per_w // chunk
    nchunks = ntch * G
    mesh = plsc.VectorSubcoreMesh(core_axis_name="c", subcore_axis_name="s")

    @functools.partial(
        pl.kernel,
        mesh=mesh,
        out_type=jax.ShapeDtypeStruct((n_tok, G * VDH), jnp.float32),
        scratch_types=[
            pltpu.VMEM((G, tok_per_w), jnp.int32),
            pltpu.VMEM((2, chunk, VDH), jnp.float32),
            pltpu.SemaphoreType.DMA,
            pltpu.SemaphoreType.DMA,
        ],
    )
    def gather(table_hbm, idx_hbm, out_hbm, idx_v, rows_v, gsem0, gsem1):
        wid = lax.axis_index("s") * info.num_cores + lax.axis_index("c")
        t0 = wid * tok_per_w
        i_blk = t0 // BLK
        off = t0 % BLK
        pltpu.sync_copy(idx_hbm.at[i_blk, :, pl.ds(off, tok_per_w)], idx_v)
        gsems = (gsem0, gsem1)

        def start(c, b):
            j, g = c // G, c % G
            return pltpu.async_copy(
                table_hbm.at[idx_v.at[g, pl.ds(j * chunk, chunk)]],
                rows_v.at[b],
                gsems[b],
            )

        cps = [start(0, 0), None]
        for c in range(nchunks):
            b = c % 2
            j, g = c // G, c % G
            cps[b].wait()
            if c + 1 < nchunks:
                cps[1 - b] = start(c + 1, 1 - b)
            pltpu.sync_copy(
                rows_v.at[b],
                out_hbm.at[pl.ds(t0 + j * chunk, chunk), pl.ds(g * VDH, VDH)],
            )

    return gather


def kernel(x, W1, b1, W2, b2, codebook):
    bsz, tsz, fsz = x.shape
    xf = x.reshape(-1, fsz)
    n = xf.shape[0]
    idx = _tc_mlp_argmax(xf, W1, b1.reshape(1, D), W2, b2.reshape(1, G * NV))
    table_bf = codebook.reshape(G * NV, VD).astype(jnp.bfloat16)
    table = lax.bitcast_convert_type(
        table_bf.reshape(G * NV, VDH, 2), jnp.float32
    )
    rows = _make_sc_gather(n)(table, idx)
    rows_bf = lax.bitcast_convert_type(rows, jnp.bfloat16)
    return rows_bf.astype(jnp.float32).reshape(bsz, tsz, G * VD)


# speedup vs baseline: 1.8776x; 1.8776x over previous
import functools

import jax
import jax.numpy as jnp
from jax import lax
from jax.experimental import pallas as pl
from jax.experimental.pallas import tpu as pltpu
from jax.experimental.pallas import tpu_sc as plsc

G = 2
NV = 320
D = 1024
VD = 512
VDH = VD // 2

BLK = 512


def _mlp_argmax_body(x_ref, w1_ref, b1_ref, w2_ref, b2_ref, idx_ref):
    x = x_ref[...]
    h = lax.dot_general(x, w1_ref[...], (((1,), (1,)), ((), ())),
                        preferred_element_type=jnp.float32)
    h = h + b1_ref[...]
    h = h * 0.5 * (1.0 + lax.erf(h * jnp.float32(0.7071067811865476)))
    logits = lax.dot_general(h, w2_ref[...], (((1,), (1,)), ((), ())),
                             preferred_element_type=jnp.float32)
    logits = logits + b2_ref[...]
    col = lax.broadcasted_iota(jnp.int32, (BLK, G * NV), 1)
    neg = jnp.float32(-jnp.inf)
    big = jnp.int32(2**30)
    outs = []
    for g in range(G):
        mask = (col >= g * NV) & (col < (g + 1) * NV)
        m = jnp.max(jnp.where(mask, logits, neg), axis=1, keepdims=True)
        hit = (logits == m) & mask
        outs.append(jnp.min(jnp.where(hit, col, big), axis=1))
    idx_ref[0] = jnp.stack(outs)


def _tc_mlp_argmax(xf, w1, b1, w2, b2):
    n = xf.shape[0]
    nblk = n // BLK
    return pl.pallas_call(
        _mlp_argmax_body,
        grid=(nblk,),
        in_specs=[
            pl.BlockSpec((BLK, D), lambda i: (i, 0)),
            pl.BlockSpec((D, D), lambda i: (0, 0)),
            pl.BlockSpec((1, D), lambda i: (0, 0)),
            pl.BlockSpec((G * NV, D), lambda i: (0, 0)),
            pl.BlockSpec((1, G * NV), lambda i: (0, 0)),
        ],
        out_specs=pl.BlockSpec((1, G, BLK), lambda i: (i, 0, 0)),
        out_shape=jax.ShapeDtypeStruct((nblk, G, BLK), jnp.int32),
    )(xf, w1, b1, w2, b2)


def _make_sc_gather(n_tok):
    info = plsc.get_sparse_core_info()
    nw = info.num_cores * info.num_subcores
    tok_per_w = n_tok // nw
    chunk = 128
    ntch = tok_per_w // chunk
    nchunks = ntch * G
    mesh = plsc.VectorSubcoreMesh(core_axis_name="c", subcore_axis_name="s")

    @functools.partial(
        pl.kernel,
        mesh=mesh,
        out_type=jax.ShapeDtypeStruct((n_tok, G * VDH), jnp.float32),
        scratch_types=[
            pltpu.VMEM((G, tok_per_w), jnp.int32),
            pltpu.VMEM((2, chunk, VDH), jnp.float32),
            pltpu.SemaphoreType.DMA,
            pltpu.SemaphoreType.DMA,
        ],
    )
    def gather(table_hbm, idx_hbm, out_hbm, idx_v, rows_v, gsem0, gsem1):
        wid = lax.axis_index("s") * info.num_cores + lax.axis_index("c")
        t0 = wid * tok_per_w
        i_blk = t0 // BLK
        off = t0 % BLK
        pltpu.sync_copy(idx_hbm.at[i_blk, :, pl.ds(off, tok_per_w)], idx_v)
        gsems = (gsem0, gsem1)

        def start(c, b):
            j, g = c // G, c % G
            return pltpu.async_copy(
                table_hbm.at[idx_v.at[g, pl.ds(j * chunk, chunk)]],
                rows_v.at[b],
                gsems[b],
            )

        cps = [start(0, 0), None]
        for c in range(nchunks):
            b = c % 2
            j, g = c // G, c % G
            cps[b].wait()
            if c + 1 < nchunks:
                cps[1 - b] = start(c + 1, 1 - b)
            pltpu.sync_copy(
                rows_v.at[b],
                out_hbm.at[pl.ds(t0 + j * chunk, chunk), pl.ds(g * VDH, VDH)],
            )

    return gather


def _unpack_body(p_ref, o_ref):
    xi = lax.bitcast_convert_type(p_ref[...], jnp.uint32)
    lo = lax.bitcast_convert_type(xi << jnp.uint32(16), jnp.float32)
    hi = lax.bitcast_convert_type(xi & jnp.uint32(0xFFFF0000), jnp.float32)
    for g in range(G):
        o_ref[:, g * VD:g * VD + VDH] = lo[:, g * VDH:(g + 1) * VDH]
        o_ref[:, g * VD + VDH:(g + 1) * VD] = hi[:, g * VDH:(g + 1) * VDH]


def _tc_unpack(packed):
    n = packed.shape[0]
    nblk = n // BLK
    return pl.pallas_call(
        _unpack_body,
        grid=(nblk,),
        in_specs=[pl.BlockSpec((BLK, G * VDH), lambda i: (i, 0))],
        out_specs=pl.BlockSpec((BLK, G * VD), lambda i: (i, 0)),
        out_shape=jax.ShapeDtypeStruct((n, G * VD), jnp.float32),
    )(packed)


def _pack_table(codebook):
    tb = codebook.reshape(G * NV, VD).astype(jnp.bfloat16)
    u = lax.bitcast_convert_type(tb, jnp.uint16).astype(jnp.uint32)
    w = u[:, :VDH] | (u[:, VDH:] << jnp.uint32(16))
    return lax.bitcast_convert_type(w, jnp.float32)


def kernel(x, W1, b1, W2, b2, codebook):
    bsz, tsz, fsz = x.shape
    xf = x.reshape(-1, fsz)
    n = xf.shape[0]
    idx = _tc_mlp_argmax(xf, W1, b1.reshape(1, D), W2, b2.reshape(1, G * NV))
    table = _pack_table(codebook)
    rows = _make_sc_gather(n)(table, idx)
    return _tc_unpack(rows).reshape(bsz, tsz, G * VD)


# speedup vs baseline: 1.9298x; 1.0278x over previous
import functools

import jax
import jax.numpy as jnp
from jax import lax
from jax.experimental import pallas as pl
from jax.experimental.pallas import tpu as pltpu
from jax.experimental.pallas import tpu_sc as plsc

G = 2
NV = 320
D = 1024
VD = 512
VDH = VD // 2

BLK = 1024


def _mlp_argmax_body(x_ref, w1_ref, b1_ref, w2_ref, b2_ref, idx_ref):
    x = x_ref[...]
    h = lax.dot_general(x, w1_ref[...], (((1,), (1,)), ((), ())),
                        preferred_element_type=jnp.float32)
    h = h + b1_ref[...]
    h = h * 0.5 * (1.0 + lax.erf(h * jnp.float32(0.7071067811865476)))
    logits = lax.dot_general(h, w2_ref[...], (((1,), (1,)), ((), ())),
                             preferred_element_type=jnp.float32)
    logits = logits + b2_ref[...]
    col = lax.broadcasted_iota(jnp.int32, (BLK, G * NV), 1)
    neg = jnp.float32(-jnp.inf)
    big = jnp.int32(2**30)
    outs = []
    for g in range(G):
        mask = (col >= g * NV) & (col < (g + 1) * NV)
        m = jnp.max(jnp.where(mask, logits, neg), axis=1, keepdims=True)
        hit = (logits == m) & mask
        outs.append(jnp.min(jnp.where(hit, col, big), axis=1))
    idx_ref[0] = jnp.stack(outs)


def _tc_mlp_argmax(xf, w1, b1, w2, b2):
    n = xf.shape[0]
    nblk = n // BLK
    return pl.pallas_call(
        _mlp_argmax_body,
        grid=(nblk,),
        in_specs=[
            pl.BlockSpec((BLK, D), lambda i: (i, 0)),
            pl.BlockSpec((D, D), lambda i: (0, 0)),
            pl.BlockSpec((1, D), lambda i: (0, 0)),
            pl.BlockSpec((G * NV, D), lambda i: (0, 0)),
            pl.BlockSpec((1, G * NV), lambda i: (0, 0)),
        ],
        out_specs=pl.BlockSpec((1, G, BLK), lambda i: (i, 0, 0)),
        out_shape=jax.ShapeDtypeStruct((nblk, G, BLK), jnp.int32),
    )(xf, w1, b1, w2, b2)


def _make_sc_gather(n_tok):
    info = plsc.get_sparse_core_info()
    nw = info.num_cores * info.num_subcores
    tok_per_w = n_tok // nw
    chunk = 128
    ntch = tok_per_w // chunk
    nchunks = ntch * G
    mesh = plsc.VectorSubcoreMesh(core_axis_name="c", subcore_axis_name="s")

    @functools.partial(
        pl.kernel,
        mesh=mesh,
        out_type=jax.ShapeDtypeStruct((n_tok, G * VDH), jnp.float32),
        scratch_types=[
            pltpu.VMEM((G, tok_per_w), jnp.int32),
            pltpu.VMEM((2, chunk, VDH), jnp.float32),
            pltpu.SemaphoreType.DMA,
            pltpu.SemaphoreType.DMA,
        ],
    )
    def gather(table_hbm, idx_hbm, out_hbm, idx_v, rows_v, gsem0, gsem1):
        wid = lax.axis_index("s") * info.num_cores + lax.axis_index("c")
        t0 = wid * tok_per_w
        i_blk = t0 // BLK
        off = t0 % BLK
        pltpu.sync_copy(idx_hbm.at[i_blk, :, pl.ds(off, tok_per_w)], idx_v)
        gsems = (gsem0, gsem1)

        def start(c, b):
            j, g = c // G, c % G
            return pltpu.async_copy(
                table_hbm.at[idx_v.at[g, pl.ds(j * chunk, chunk)]],
                rows_v.at[b],
                gsems[b],
            )

        cps = [start(0, 0), None]
        for c in range(nchunks):
            b = c % 2
            j, g = c // G, c % G
            cps[b].wait()
            if c + 1 < nchunks:
                cps[1 - b] = start(c + 1, 1 - b)
            pltpu.sync_copy(
                rows_v.at[b],
                out_hbm.at[pl.ds(t0 + j * chunk, chunk), pl.ds(g * VDH, VDH)],
            )

    return gather


def _unpack_body(p_ref, o_ref):
    xi = lax.bitcast_convert_type(p_ref[...], jnp.uint32)
    lo = lax.bitcast_convert_type(xi << jnp.uint32(16), jnp.float32)
    hi = lax.bitcast_convert_type(xi & jnp.uint32(0xFFFF0000), jnp.float32)
    for g in range(G):
        o_ref[:, g * VD:g * VD + VDH] = lo[:, g * VDH:(g + 1) * VDH]
        o_ref[:, g * VD + VDH:(g + 1) * VD] = hi[:, g * VDH:(g + 1) * VDH]


def _tc_unpack(packed):
    n = packed.shape[0]
    nblk = n // BLK
    return pl.pallas_call(
        _unpack_body,
        grid=(nblk,),
        in_specs=[pl.BlockSpec((BLK, G * VDH), lambda i: (i, 0))],
        out_specs=pl.BlockSpec((BLK, G * VD), lambda i: (i, 0)),
        out_shape=jax.ShapeDtypeStruct((n, G * VD), jnp.float32),
    )(packed)


def _pack_table(codebook):
    tb = codebook.reshape(G * NV, VD).astype(jnp.bfloat16)
    u = lax.bitcast_convert_type(tb, jnp.uint16).astype(jnp.uint32)
    w = u[:, :VDH] | (u[:, VDH:] << jnp.uint32(16))
    return lax.bitcast_convert_type(w, jnp.float32)


def kernel(x, W1, b1, W2, b2, codebook):
    bsz, tsz, fsz = x.shape
    xf = x.reshape(-1, fsz)
    n = xf.shape[0]
    idx = _tc_mlp_argmax(xf, W1, b1.reshape(1, D), W2, b2.reshape(1, G * NV))
    table = _pack_table(codebook)
    rows = _make_sc_gather(n)(table, idx)
    return _tc_unpack(rows).reshape(bsz, tsz, G * VD)


# speedup vs baseline: 2.0127x; 1.0430x over previous
import functools

import jax
import jax.numpy as jnp
from jax import lax
from jax.experimental import pallas as pl
from jax.experimental.pallas import tpu as pltpu
from jax.experimental.pallas import tpu_sc as plsc

G = 2
NV = 320
D = 1024
VD = 512
VDH = VD // 2

BLK = 1024


def _mlp_argmax_body(x_ref, w1_ref, b1_ref, w2_ref, b2_ref, idx_ref):
    x = x_ref[...]
    h = lax.dot_general(x, w1_ref[...], (((1,), (1,)), ((), ())),
                        preferred_element_type=jnp.float32)
    h = h + b1_ref[...]
    h = h * 0.5 * (1.0 + lax.erf(h * jnp.float32(0.7071067811865476)))
    logits = lax.dot_general(h, w2_ref[...], (((1,), (1,)), ((), ())),
                             preferred_element_type=jnp.float32)
    logits = logits + b2_ref[...]
    col = lax.broadcasted_iota(jnp.int32, (BLK, G * NV), 1)
    neg = jnp.float32(-jnp.inf)
    big = jnp.int32(2**30)
    outs = []
    for g in range(G):
        mask = (col >= g * NV) & (col < (g + 1) * NV)
        m = jnp.max(jnp.where(mask, logits, neg), axis=1, keepdims=True)
        hit = (logits == m) & mask
        outs.append(jnp.min(jnp.where(hit, col, big), axis=1))
    idx_ref[0] = jnp.stack(outs)


def _tc_mlp_argmax(xf, w1, b1, w2, b2):
    n = xf.shape[0]
    nblk = n // BLK
    return pl.pallas_call(
        _mlp_argmax_body,
        grid=(nblk,),
        in_specs=[
            pl.BlockSpec((BLK, D), lambda i: (i, 0)),
            pl.BlockSpec((D, D), lambda i: (0, 0)),
            pl.BlockSpec((1, D), lambda i: (0, 0)),
            pl.BlockSpec((G * NV, D), lambda i: (0, 0)),
            pl.BlockSpec((1, G * NV), lambda i: (0, 0)),
        ],
        out_specs=pl.BlockSpec((1, G, BLK), lambda i: (i, 0, 0)),
        out_shape=jax.ShapeDtypeStruct((nblk, G, BLK), jnp.int32),
    )(xf, w1, b1, w2, b2)


def _make_sc_gather(n_tok):
    info = plsc.get_sparse_core_info()
    nw = info.num_cores * info.num_subcores
    tok_per_w = n_tok // nw
    chunk = 64
    ntch = tok_per_w // chunk
    nchunks = ntch * G
    nbuf = 4
    mesh = plsc.VectorSubcoreMesh(core_axis_name="c", subcore_axis_name="s")

    @functools.partial(
        pl.kernel,
        mesh=mesh,
        out_type=jax.ShapeDtypeStruct((n_tok, G * VDH), jnp.float32),
        scratch_types=[
            pltpu.VMEM((G, tok_per_w), jnp.int32),
            pltpu.VMEM((nbuf, chunk, VDH), jnp.float32),
            pltpu.SemaphoreType.DMA((nbuf,)),
            pltpu.SemaphoreType.DMA((nbuf,)),
        ],
    )
    def gather(table_hbm, idx_hbm, out_hbm, idx_v, rows_v, gsem, ssem):
        wid = lax.axis_index("s") * info.num_cores + lax.axis_index("c")
        t0 = wid * tok_per_w
        i_blk = t0 // BLK
        off = t0 % BLK
        pltpu.sync_copy(idx_hbm.at[i_blk, :, pl.ds(off, tok_per_w)], idx_v)

        def start(c):
            j, g = c // G, c % G
            return pltpu.async_copy(
                table_hbm.at[idx_v.at[g, pl.ds(j * chunk, chunk)]],
                rows_v.at[c % nbuf],
                gsem.at[c % nbuf],
            )

        def store(c):
            j, g = c // G, c % G
            return pltpu.async_copy(
                rows_v.at[c % nbuf],
                out_hbm.at[pl.ds(t0 + j * chunk, chunk), pl.ds(g * VDH, VDH)],
                ssem.at[c % nbuf],
            )

        gcp = [None] * nchunks
        scp = [None] * nchunks
        gcp[0] = start(0)
        if nchunks > 1:
            gcp[1] = start(1)
        for c in range(nchunks):
            if c + 2 < nchunks:
                if c - 2 >= 0:
                    scp[c - 2].wait()
                gcp[c + 2] = start(c + 2)
            gcp[c].wait()
            scp[c] = store(c)
        for c in range(max(0, nchunks - 4), nchunks):
            scp[c].wait()

    return gather


def _unpack_body(p_ref, o_ref):
    xi = lax.bitcast_convert_type(p_ref[...], jnp.uint32)
    lo = lax.bitcast_convert_type(xi << jnp.uint32(16), jnp.float32)
    hi = lax.bitcast_convert_type(xi & jnp.uint32(0xFFFF0000), jnp.float32)
    for g in range(G):
        o_ref[:, g * VD:g * VD + VDH] = lo[:, g * VDH:(g + 1) * VDH]
        o_ref[:, g * VD + VDH:(g + 1) * VD] = hi[:, g * VDH:(g + 1) * VDH]


def _tc_unpack(packed):
    n = packed.shape[0]
    nblk = n // BLK
    return pl.pallas_call(
        _unpack_body,
        grid=(nblk,),
        in_specs=[pl.BlockSpec((BLK, G * VDH), lambda i: (i, 0))],
        out_specs=pl.BlockSpec((BLK, G * VD), lambda i: (i, 0)),
        out_shape=jax.ShapeDtypeStruct((n, G * VD), jnp.float32),
    )(packed)


def _pack_table(codebook):
    tb = codebook.reshape(G * NV, VD).astype(jnp.bfloat16)
    u = lax.bitcast_convert_type(tb, jnp.uint16).astype(jnp.uint32)
    w = u[:, :VDH] | (u[:, VDH:] << jnp.uint32(16))
    return lax.bitcast_convert_type(w, jnp.float32)


def kernel(x, W1, b1, W2, b2, codebook):
    bsz, tsz, fsz = x.shape
    xf = x.reshape(-1, fsz)
    n = xf.shape[0]
    idx = _tc_mlp_argmax(xf, W1, b1.reshape(1, D), W2, b2.reshape(1, G * NV))
    table = _pack_table(codebook)
    rows = _make_sc_gather(n)(table, idx)
    return _tc_unpack(rows).reshape(bsz, tsz, G * VD)
